# baseline (speedup 1.0000x reference)
import functools
import jax
import jax.numpy as jnp
import numpy as np
from jax.experimental import pallas as pl
from jax.experimental.pallas import tpu as pltpu

N = 10000
NB = 128
NI = 3
NRBF = 20
CUTOFF = 5.0
MAXZ = 100
BOX = 54.7
EPS = 1e-08
E_MAX = 524288

_NRBF_PAD = 24
_TE = 512


def _filter_block(d_ref, m_ref, W_ref, b_ref, out_ref):
    d = d_ref[:][:, None]
    width = CUTOFF / (NRBF - 1)
    off = jax.lax.broadcasted_iota(
        jnp.int32, (1, _NRBF_PAD), 1).astype(jnp.float32) * width
    phi = jnp.exp(-0.5 * ((d - off) / width) ** 2)
    fcut = 0.5 * (jnp.cos(d * (np.pi / CUTOFF)) + 1.0) * (d < CUTOFF)
    f = jnp.dot(phi, W_ref[:, :], preferred_element_type=jnp.float32)
    f = (f + b_ref[:][None, :]) * fcut * m_ref[:][:, None]
    out_ref[:, :] = f


def _filters_pallas(d, m, filter_W, filter_b):
    W_pad = jnp.zeros((_NRBF_PAD, NI * 3 * NB), jnp.float32)
    W_pad = W_pad.at[:NRBF].set(filter_W)
    grid = (E_MAX // _TE,)
    return pl.pallas_call(
        _filter_block,
        grid=grid,
        in_specs=[
            pl.BlockSpec((_TE,), lambda i: (i,)),
            pl.BlockSpec((_TE,), lambda i: (i,)),
            pl.BlockSpec((_NRBF_PAD, NI * 3 * NB), lambda i: (0, 0)),
            pl.BlockSpec((NI * 3 * NB,), lambda i: (0,)),
        ],
        out_specs=pl.BlockSpec((_TE, NI * 3 * NB), lambda i: (i, 0)),
        out_shape=jax.ShapeDtypeStruct((E_MAX, NI * 3 * NB), jnp.float32),
    )(d, m, W_pad, filter_b)


def _radius_graph(pos):
    d2 = jnp.sum(pos * pos, axis=1)
    D2 = d2[:, None] + d2[None, :] - 2.0 * (pos @ pos.T)
    mask = D2 < CUTOFF ** 2
    ar = jnp.arange(N)
    mask = mask.at[ar, ar].set(False)
    src, dst = jnp.nonzero(mask, size=E_MAX, fill_value=0)
    edge_mask = jnp.arange(E_MAX) < jnp.sum(mask)
    idx_j, idx_i = src, dst
    return idx_i, idx_j, edge_mask


def kernel(z, pos, emb, filter_W, filter_b, inter_W1, inter_b1, inter_W2,
           inter_b2, mix_W1, mix_b1, mix_W2, mix_b2, mu_mix_W):
    idx_i, idx_j, edge_mask = _radius_graph(pos)
    r_ij = pos[idx_j] - pos[idx_i]
    r_ij = jnp.where(jnp.abs(r_ij) <= 1e-06, 1e-06, r_ij)
    d_ij = jnp.sqrt(jnp.sum(r_ij * r_ij, axis=1))
    dir_ij = r_ij / d_ij[:, None]

    filters = _filters_pallas(d_ij, edge_mask.astype(jnp.float32),
                              filter_W, filter_b)

    q = emb[z][:, None, :]
    mu = jnp.zeros((N, 3, NB), jnp.float32)
    for i in range(NI):
        x = jax.nn.silu(q @ inter_W1[i] + inter_b1[i])
        x = x @ inter_W2[i] + inter_b2[i]
        xj = x[idx_j]
        muj = mu[idx_j]
        xf = filters[:, None, i * 3 * NB:(i + 1) * 3 * NB] * xj
        dq, dmuR, dmumu = jnp.split(xf, 3, axis=-1)
        dq = jnp.zeros((N, 1, NB), jnp.float32).at[idx_i].add(dq)
        dmu = dmuR * dir_ij[:, :, None] + dmumu * muj
        dmu = jnp.zeros((N, 3, NB), jnp.float32).at[idx_i].add(dmu)
        q = q + dq
        mu = mu + dmu
        mu_mix = mu @ mu_mix_W[i]
        mu_V, mu_Wc = jnp.split(mu_mix, 2, axis=-1)
        mu_Vn = jnp.sqrt(jnp.sum(mu_V ** 2, axis=-2, keepdims=True) + EPS)
        ctx = jnp.concatenate([q, mu_Vn], axis=-1)
        xm = jax.nn.silu(ctx @ mix_W1[i] + mix_b1[i])
        xm = xm @ mix_W2[i] + mix_b2[i]
        dq_intra, dmu_intra, dqmu_intra = jnp.split(xm, 3, axis=-1)
        dmu_intra = dmu_intra * mu_Wc
        dqmu_intra = dqmu_intra * jnp.sum(mu_V * mu_Wc, axis=1, keepdims=True)
        q = q + dq_intra + dqmu_intra
        mu = mu + dmu_intra
    return q[:, 0, :], mu


# speedup vs baseline: 3.7849x; 3.7849x over previous
import functools
import jax
import jax.numpy as jnp
import numpy as np
from jax import lax
from jax.experimental import pallas as pl
from jax.experimental.pallas import tpu as pltpu
from jax.experimental.pallas import tpu_sc as plsc

N = 10000
NB = 128
NI = 3
NRBF = 20
CUTOFF = 5.0
BOX = 54.7
EPS = 1e-08
E_MAX = 524288

_NRBF_PAD = 24
_T = 256
_W = 512
_NPAD = 10240
_NTILES = E_MAX // _T

_NWORKERS = 32
_BPW = E_MAX // _NWORKERS
_GC = 128


def _sc_gather_body(tab_hbm, idx_hbm, out_hbm, idxc_v, rows_v, sem):
    c = lax.axis_index("c")
    s = lax.axis_index("s")
    wid = s * 2 + c
    base = wid * _BPW

    def body(i, carry):
        off = base + i * _GC
        pltpu.sync_copy(idx_hbm.at[pl.ds(off, _GC)], idxc_v)
        pltpu.async_copy(tab_hbm.at[idxc_v], rows_v, sem).wait()
        pltpu.sync_copy(rows_v, out_hbm.at[pl.ds(off, _GC)])
        return carry

    lax.fori_loop(0, _BPW // _GC, body, 0)


def _sc_gather(tab, idx):
    mesh = plsc.VectorSubcoreMesh(core_axis_name="c", subcore_axis_name="s")
    fn = pl.kernel(
        _sc_gather_body,
        out_type=jax.ShapeDtypeStruct((E_MAX, 6 * NB), jnp.float32),
        mesh=mesh,
        scratch_types=[
            pltpu.VMEM((_GC,), jnp.int32),
            pltpu.VMEM((_GC, 6 * NB), jnp.float32),
            pltpu.SemaphoreType.DMA,
        ],
    )
    return fn(tab, idx)


def _msg_block(base_ref, d_ref, m_ref, dx_ref, dy_ref, dz_ref, dst_ref,
               gath_ref, W_ref, b_ref, out_ref):
    t = pl.program_id(0)

    @pl.when(t == 0)
    def _():
        out_ref[:, :] = jnp.zeros_like(out_ref)

    d = d_ref[:][:, None]
    width = CUTOFF / (NRBF - 1)
    off = lax.broadcasted_iota(jnp.int32, (1, _NRBF_PAD), 1).astype(
        jnp.float32) * width
    phi = jnp.exp(-0.5 * ((d - off) / width) ** 2)
    fcut = 0.5 * (jnp.cos(d * (np.pi / CUTOFF)) + 1.0) * (d < CUTOFF)
    filt = jnp.dot(phi, W_ref[:, :], preferred_element_type=jnp.float32)
    filt = (filt + b_ref[:][None, :]) * fcut * m_ref[:][:, None]

    xj = gath_ref[:, :3 * NB]
    xf = filt * xj
    f1 = xf[:, :NB]
    f2 = xf[:, NB:2 * NB]
    f3 = xf[:, 2 * NB:3 * NB]
    mux = gath_ref[:, 3 * NB:4 * NB]
    muy = gath_ref[:, 4 * NB:5 * NB]
    muz = gath_ref[:, 5 * NB:6 * NB]
    contrib = jnp.concatenate([
        f1,
        f2 * dx_ref[:][:, None] + f3 * mux,
        f2 * dy_ref[:][:, None] + f3 * muy,
        f2 * dz_ref[:][:, None] + f3 * muz,
    ], axis=1)

    base = pl.multiple_of(base_ref[t], 8)
    rel = dst_ref[:] - base
    oh = (lax.broadcasted_iota(jnp.int32, (_W, _T), 0)
          == rel[None, :]).astype(jnp.float32)
    upd = jnp.dot(oh, contrib, preferred_element_type=jnp.float32)
    out_ref[pl.ds(base, _W), :] += upd


def _msg_pallas(node_base, d, m, dx, dy, dz, dst, gath, W_slice, b_slice):
    return pl.pallas_call(
        _msg_block,
        grid=(_NTILES,),
        in_specs=[
            pl.BlockSpec(memory_space=pltpu.SMEM),
            pl.BlockSpec((_T,), lambda i: (i,)),
            pl.BlockSpec((_T,), lambda i: (i,)),
            pl.BlockSpec((_T,), lambda i: (i,)),
            pl.BlockSpec((_T,), lambda i: (i,)),
            pl.BlockSpec((_T,), lambda i: (i,)),
            pl.BlockSpec((_T,), lambda i: (i,)),
            pl.BlockSpec((_T, 6 * NB), lambda i: (i, 0)),
            pl.BlockSpec((_NRBF_PAD, 3 * NB), lambda i: (0, 0)),
            pl.BlockSpec((3 * NB,), lambda i: (0,)),
        ],
        out_specs=pl.BlockSpec((_NPAD, 4 * NB), lambda i: (0, 0)),
        out_shape=jax.ShapeDtypeStruct((_NPAD, 4 * NB), jnp.float32),
    )(node_base, d, m, dx, dy, dz, dst, gath, W_slice, b_slice)


def kernel(z, pos, emb, filter_W, filter_b, inter_W1, inter_b1, inter_W2,
           inter_b2, mix_W1, mix_b1, mix_W2, mix_b2, mu_mix_W):
    d2n = jnp.sum(pos * pos, axis=1)
    D2 = d2n[:, None] + d2n[None, :] - 2.0 * (pos @ pos.T)
    mask = D2 < CUTOFF ** 2
    ar = jnp.arange(N)
    mask = mask.at[ar, ar].set(False)
    idx_i, idx_j = jnp.nonzero(mask, size=E_MAX, fill_value=0)
    edge_mask = jnp.arange(E_MAX) < jnp.sum(mask)
    m_f = edge_mask.astype(jnp.float32)

    r_ij = pos[idx_j] - pos[idx_i]
    r_ij = jnp.where(jnp.abs(r_ij) <= 1e-06, 1e-06, r_ij)
    d_ij = jnp.sqrt(jnp.sum(r_ij * r_ij, axis=1))
    dir_ij = r_ij / d_ij[:, None]
    dx, dy, dz = dir_ij[:, 0], dir_ij[:, 1], dir_ij[:, 2]

    dst = idx_i.astype(jnp.int32)
    src = idx_j.astype(jnp.int32)
    node_base = jnp.minimum((dst[::_T] // 8) * 8, _NPAD - _W).astype(jnp.int32)

    W_pad = jnp.zeros((_NRBF_PAD, NI * 3 * NB), jnp.float32)
    W_pad = W_pad.at[:NRBF].set(filter_W)

    q = emb[z]
    mu = jnp.zeros((N, 3 * NB), jnp.float32)
    for i in range(NI):
        x = jax.nn.silu(q @ inter_W1[i] + inter_b1[i])
        x = x @ inter_W2[i] + inter_b2[i]
        tab = jnp.concatenate([x, mu], axis=1)
        gath = _sc_gather(tab, src)
        acc = _msg_pallas(node_base, d_ij, m_f, dx, dy, dz, dst, gath,
                          W_pad[:, i * 3 * NB:(i + 1) * 3 * NB],
                          filter_b[i * 3 * NB:(i + 1) * 3 * NB])
        q = q + acc[:N, :NB]
        mu = mu + acc[:N, NB:]
        mu3 = mu.reshape(N, 3, NB)
        mu_mix = mu3 @ mu_mix_W[i]
        mu_V, mu_Wc = jnp.split(mu_mix, 2, axis=-1)
        mu_Vn = jnp.sqrt(jnp.sum(mu_V ** 2, axis=-2) + EPS)
        ctx = jnp.concatenate([q, mu_Vn], axis=-1)
        xm = jax.nn.silu(ctx @ mix_W1[i] + mix_b1[i])
        xm = xm @ mix_W2[i] + mix_b2[i]
        dq_intra = xm[:, :NB]
        dmu_intra = xm[:, None, NB:2 * NB] * mu_Wc
        dqmu_intra = xm[:, 2 * NB:] * jnp.sum(mu_V * mu_Wc, axis=1)
        q = q + dq_intra + dqmu_intra
        mu = (mu3 + dmu_intra).reshape(N, 3 * NB)
    return q, mu.reshape(N, 3, NB)


# speedup vs baseline: 3.8837x; 1.0261x over previous
import functools
import jax
import jax.numpy as jnp
import numpy as np
from jax import lax
from jax.experimental import pallas as pl
from jax.experimental.pallas import tpu as pltpu
from jax.experimental.pallas import tpu_sc as plsc

N = 10000
NB = 128
NI = 3
NRBF = 20
CUTOFF = 5.0
BOX = 54.7
EPS = 1e-08
E_MAX = 524288

_NRBF_PAD = 24
_T = 256
_W = 512
_NPAD = 10240
_NTILES = E_MAX // _T

_NWORKERS = 32
_BPW = E_MAX // _NWORKERS
_GC = 64


def _sc_gather_body(tab_hbm, idx_hbm, out_hbm, idx_v, rows0, rows1,
                    sg0, sg1, sw0, sw1):
    c = lax.axis_index("c")
    s = lax.axis_index("s")
    wid = s * 2 + c
    base = wid * _BPW
    pltpu.sync_copy(idx_hbm.at[pl.ds(base, _BPW)], idx_v)

    nchunks = _BPW // _GC
    kmax = nchunks // 2

    def _gather(chunk, rows, sem):
        pltpu.async_copy(tab_hbm.at[idx_v.at[pl.ds(chunk * _GC, _GC)]],
                         rows, sem)

    def _wb(chunk, rows, sem):
        pltpu.async_copy(rows, out_hbm.at[pl.ds(base + chunk * _GC, _GC)],
                         sem)

    def _drain(rows, sem):
        pltpu.make_async_copy(out_hbm.at[pl.ds(0, _GC)], rows, sem).wait()

    _gather(0, rows0, sg0)

    def body(k, carry):
        c0 = 2 * k
        _drain(rows0, sg0)
        _gather(c0 + 1, rows1, sg1)
        _wb(c0, rows0, sw0)
        _drain(rows0, sw0)
        @pl.when(k < kmax - 1)
        def _():
            _gather(c0 + 2, rows0, sg0)
        _drain(rows1, sg1)
        _wb(c0 + 1, rows1, sw1)
        _drain(rows1, sw1)
        return carry

    lax.fori_loop(0, kmax, body, 0)


def _sc_gather(tab, idx):
    mesh = plsc.VectorSubcoreMesh(core_axis_name="c", subcore_axis_name="s")
    fn = pl.kernel(
        _sc_gather_body,
        out_type=jax.ShapeDtypeStruct((E_MAX, 6 * NB), jnp.float32),
        mesh=mesh,
        scratch_types=[
            pltpu.VMEM((_BPW,), jnp.int32),
            pltpu.VMEM((_GC, 6 * NB), jnp.float32),
            pltpu.VMEM((_GC, 6 * NB), jnp.float32),
            pltpu.SemaphoreType.DMA,
            pltpu.SemaphoreType.DMA,
            pltpu.SemaphoreType.DMA,
            pltpu.SemaphoreType.DMA,
        ],
    )
    return fn(tab, idx)


def _msg_block(base_ref, d_ref, m_ref, dx_ref, dy_ref, dz_ref, dst_ref,
               gath_ref, W_ref, b_ref, out_ref):
    t = pl.program_id(0)

    @pl.when(t == 0)
    def _():
        out_ref[:, :] = jnp.zeros_like(out_ref)

    d = d_ref[:][:, None]
    width = CUTOFF / (NRBF - 1)
    off = lax.broadcasted_iota(jnp.int32, (1, _NRBF_PAD), 1).astype(
        jnp.float32) * width
    phi = jnp.exp(-0.5 * ((d - off) / width) ** 2)
    fcut = 0.5 * (jnp.cos(d * (np.pi / CUTOFF)) + 1.0) * (d < CUTOFF)
    filt = jnp.dot(phi, W_ref[:, :], preferred_element_type=jnp.float32)
    filt = (filt + b_ref[:][None, :]) * fcut * m_ref[:][:, None]

    xj = gath_ref[:, :3 * NB]
    xf = filt * xj
    f1 = xf[:, :NB]
    f2 = xf[:, NB:2 * NB]
    f3 = xf[:, 2 * NB:3 * NB]
    mux = gath_ref[:, 3 * NB:4 * NB]
    muy = gath_ref[:, 4 * NB:5 * NB]
    muz = gath_ref[:, 5 * NB:6 * NB]
    contrib = jnp.concatenate([
        f1,
        f2 * dx_ref[:][:, None] + f3 * mux,
        f2 * dy_ref[:][:, None] + f3 * muy,
        f2 * dz_ref[:][:, None] + f3 * muz,
    ], axis=1)

    base = pl.multiple_of(base_ref[t], 8)
    rel = dst_ref[:] - base
    oh = (lax.broadcasted_iota(jnp.int32, (_W, _T), 0)
          == rel[None, :]).astype(jnp.float32)
    upd = jnp.dot(oh, contrib, preferred_element_type=jnp.float32)
    out_ref[pl.ds(base, _W), :] += upd


def _msg_pallas(node_base, d, m, dx, dy, dz, dst, gath, W_slice, b_slice):
    return pl.pallas_call(
        _msg_block,
        grid=(_NTILES,),
        in_specs=[
            pl.BlockSpec(memory_space=pltpu.SMEM),
            pl.BlockSpec((_T,), lambda i: (i,)),
            pl.BlockSpec((_T,), lambda i: (i,)),
            pl.BlockSpec((_T,), lambda i: (i,)),
            pl.BlockSpec((_T,), lambda i: (i,)),
            pl.BlockSpec((_T,), lambda i: (i,)),
            pl.BlockSpec((_T,), lambda i: (i,)),
            pl.BlockSpec((_T, 6 * NB), lambda i: (i, 0)),
            pl.BlockSpec((_NRBF_PAD, 3 * NB), lambda i: (0, 0)),
            pl.BlockSpec((3 * NB,), lambda i: (0,)),
        ],
        out_specs=pl.BlockSpec((_NPAD, 4 * NB), lambda i: (0, 0)),
        out_shape=jax.ShapeDtypeStruct((_NPAD, 4 * NB), jnp.float32),
    )(node_base, d, m, dx, dy, dz, dst, gath, W_slice, b_slice)


def kernel(z, pos, emb, filter_W, filter_b, inter_W1, inter_b1, inter_W2,
           inter_b2, mix_W1, mix_b1, mix_W2, mix_b2, mu_mix_W):
    d2n = jnp.sum(pos * pos, axis=1)
    D2 = d2n[:, None] + d2n[None, :] - 2.0 * (pos @ pos.T)
    ar = jnp.arange(N)
    mask = (D2 < CUTOFF ** 2) & (ar[:, None] != ar[None, :])
    idx_i, idx_j = jnp.nonzero(mask, size=E_MAX, fill_value=0)
    edge_mask = jnp.arange(E_MAX) < jnp.sum(mask)
    m_f = edge_mask.astype(jnp.float32)

    r_ij = pos[idx_j] - pos[idx_i]
    r_ij = jnp.where(jnp.abs(r_ij) <= 1e-06, 1e-06, r_ij)
    d_ij = jnp.sqrt(jnp.sum(r_ij * r_ij, axis=1))
    dir_ij = r_ij / d_ij[:, None]
    dx, dy, dz = dir_ij[:, 0], dir_ij[:, 1], dir_ij[:, 2]

    dst = idx_i.astype(jnp.int32)
    src = idx_j.astype(jnp.int32)
    node_base = jnp.minimum((dst[::_T] // 8) * 8, _NPAD - _W).astype(jnp.int32)

    W_pad = jnp.zeros((_NRBF_PAD, NI * 3 * NB), jnp.float32)
    W_pad = W_pad.at[:NRBF].set(filter_W)

    q = emb[z]
    mu = jnp.zeros((N, 3 * NB), jnp.float32)
    for i in range(NI):
        x = jax.nn.silu(q @ inter_W1[i] + inter_b1[i])
        x = x @ inter_W2[i] + inter_b2[i]
        tab = jnp.concatenate([x, mu], axis=1)
        gath = _sc_gather(tab, src)
        acc = _msg_pallas(node_base, d_ij, m_f, dx, dy, dz, dst, gath,
                          W_pad[:, i * 3 * NB:(i + 1) * 3 * NB],
                          filter_b[i * 3 * NB:(i + 1) * 3 * NB])
        q = q + acc[:N, :NB]
        mu = mu + acc[:N, NB:]
        mu3 = mu.reshape(N, 3, NB)
        mu_mix = mu3 @ mu_mix_W[i]
        mu_V, mu_Wc = jnp.split(mu_mix, 2, axis=-1)
        mu_Vn = jnp.sqrt(jnp.sum(mu_V ** 2, axis=-2) + EPS)
        ctx = jnp.concatenate([q, mu_Vn], axis=-1)
        xm = jax.nn.silu(ctx @ mix_W1[i] + mix_b1[i])
        xm = xm @ mix_W2[i] + mix_b2[i]
        dq_intra = xm[:, :NB]
        dmu_intra = xm[:, None, NB:2 * NB] * mu_Wc
        dqmu_intra = xm[:, 2 * NB:] * jnp.sum(mu_V * mu_Wc, axis=1)
        q = q + dq_intra + dqmu_intra
        mu = (mu3 + dmu_intra).reshape(N, 3 * NB)
    return q, mu.reshape(N, 3, NB)


# speedup vs baseline: 4.2704x; 1.0996x over previous
import functools
import jax
import jax.numpy as jnp
import numpy as np
from jax import lax
from jax.experimental import pallas as pl
from jax.experimental.pallas import tpu as pltpu
from jax.experimental.pallas import tpu_sc as plsc

N = 10000
NB = 128
NI = 3
NRBF = 20
CUTOFF = 5.0
BOX = 54.7
EPS = 1e-08
E_MAX = 524288

_NRBF_PAD = 24
_T = 256
_W = 512
_NPAD = 10240
_NTILES = E_MAX // _T

_NWORKERS = 32
_BPW = E_MAX // _NWORKERS
_GC = 64


def _sc_gather_body(ncols, tab_hbm, idx_hbm, out_hbm, idx_v, rows0, rows1,
                    sg0, sg1, sw0, sw1):
    c = lax.axis_index("c")
    s = lax.axis_index("s")
    wid = s * 2 + c
    base = wid * _BPW
    pltpu.sync_copy(idx_hbm.at[pl.ds(base, _BPW)], idx_v)

    nchunks = _BPW // _GC
    kmax = nchunks // 2

    def _gather(chunk, rows, sem):
        pltpu.async_copy(tab_hbm.at[idx_v.at[pl.ds(chunk * _GC, _GC)]],
                         rows, sem)

    def _wb(chunk, rows, sem):
        pltpu.async_copy(rows, out_hbm.at[pl.ds(base + chunk * _GC, _GC)],
                         sem)

    def _drain(rows, sem):
        pltpu.make_async_copy(out_hbm.at[pl.ds(0, _GC)], rows, sem).wait()

    _gather(0, rows0, sg0)

    def body(k, carry):
        c0 = 2 * k
        _drain(rows0, sg0)
        _gather(c0 + 1, rows1, sg1)
        _wb(c0, rows0, sw0)
        _drain(rows0, sw0)
        @pl.when(k < kmax - 1)
        def _():
            _gather(c0 + 2, rows0, sg0)
        _drain(rows1, sg1)
        _wb(c0 + 1, rows1, sw1)
        _drain(rows1, sw1)
        return carry

    lax.fori_loop(0, kmax, body, 0)


def _sc_gather(tab, idx):
    ncols = tab.shape[1]
    mesh = plsc.VectorSubcoreMesh(core_axis_name="c", subcore_axis_name="s")
    fn = pl.kernel(
        functools.partial(_sc_gather_body, ncols),
        out_type=jax.ShapeDtypeStruct((E_MAX, ncols), jnp.int32),
        mesh=mesh,
        scratch_types=[
            pltpu.VMEM((_BPW,), jnp.int32),
            pltpu.VMEM((_GC, ncols), jnp.int32),
            pltpu.VMEM((_GC, ncols), jnp.int32),
            pltpu.SemaphoreType.DMA,
            pltpu.SemaphoreType.DMA,
            pltpu.SemaphoreType.DMA,
            pltpu.SemaphoreType.DMA,
        ],
    )
    return fn(tab, idx)


def _pack_bf16(a):
    n, c = a.shape
    bf = a.astype(jnp.bfloat16)
    pair = jnp.stack([bf[:, :c // 2], bf[:, c // 2:]], axis=-1)
    return lax.bitcast_convert_type(pair, jnp.int32)


def _unpack_halves(w):
    lo = lax.bitcast_convert_type(w << 16, jnp.float32)
    hi = lax.bitcast_convert_type(
        w & jnp.int32(np.int32(np.uint32(0xFFFF0000))), jnp.float32)
    return lo, hi


def _msg_block(has_mu, base_ref, d_ref, m_ref, dx_ref, dy_ref, dz_ref,
               dst_ref, gath_ref, W_ref, b_ref, W2_ref, b2_ref, out_ref):
    t = pl.program_id(0)

    @pl.when(t == 0)
    def _():
        out_ref[:, :] = jnp.zeros_like(out_ref)

    d = d_ref[:][:, None]
    width = CUTOFF / (NRBF - 1)
    off = lax.broadcasted_iota(jnp.int32, (1, _NRBF_PAD), 1).astype(
        jnp.float32) * width
    phi = jnp.exp(-0.5 * ((d - off) / width) ** 2)
    fcut = 0.5 * (jnp.cos(d * (np.pi / CUTOFF)) + 1.0) * (d < CUTOFF)
    filt = jnp.dot(phi, W_ref[:, :], preferred_element_type=jnp.float32)
    filt = (filt + b_ref[:][None, :]) * fcut * m_ref[:][:, None]

    if has_mu:
        lo, hi = _unpack_halves(gath_ref[:, :])
        s = lo[:, :NB]
    else:
        s = lax.bitcast_convert_type(gath_ref[:, :], jnp.float32)
    x = jnp.dot(s, W2_ref[:, :],
                preferred_element_type=jnp.float32) + b2_ref[:][None, :]
    xf = filt * x
    f1 = xf[:, :NB]
    f2 = xf[:, NB:2 * NB]
    f3 = xf[:, 2 * NB:3 * NB]
    dmux = f2 * dx_ref[:][:, None]
    dmuy = f2 * dy_ref[:][:, None]
    dmuz = f2 * dz_ref[:][:, None]
    if has_mu:
        dmux = dmux + f3 * lo[:, NB:2 * NB]
        dmuy = dmuy + f3 * hi[:, :NB]
        dmuz = dmuz + f3 * hi[:, NB:2 * NB]
    contrib = jnp.concatenate([f1, dmux, dmuy, dmuz], axis=1)

    base = pl.multiple_of(base_ref[t], 8)
    rel = dst_ref[:] - base
    oh = (lax.broadcasted_iota(jnp.int32, (_W, _T), 0)
          == rel[None, :]).astype(jnp.float32)
    upd = jnp.dot(oh, contrib, preferred_element_type=jnp.float32)
    out_ref[pl.ds(base, _W), :] += upd


def _msg_pallas(has_mu, node_base, d, m, dx, dy, dz, dst, gath, W_slice,
                b_slice, W2, b2):
    ncols = gath.shape[1]
    return pl.pallas_call(
        functools.partial(_msg_block, has_mu),
        grid=(_NTILES,),
        in_specs=[
            pl.BlockSpec(memory_space=pltpu.SMEM),
            pl.BlockSpec((_T,), lambda i: (i,)),
            pl.BlockSpec((_T,), lambda i: (i,)),
            pl.BlockSpec((_T,), lambda i: (i,)),
            pl.BlockSpec((_T,), lambda i: (i,)),
            pl.BlockSpec((_T,), lambda i: (i,)),
            pl.BlockSpec((_T,), lambda i: (i,)),
            pl.BlockSpec((_T, ncols), lambda i: (i, 0)),
            pl.BlockSpec((_NRBF_PAD, 3 * NB), lambda i: (0, 0)),
            pl.BlockSpec((3 * NB,), lambda i: (0,)),
            pl.BlockSpec((NB, 3 * NB), lambda i: (0, 0)),
            pl.BlockSpec((3 * NB,), lambda i: (0,)),
        ],
        out_specs=pl.BlockSpec((_NPAD, 4 * NB), lambda i: (0, 0)),
        out_shape=jax.ShapeDtypeStruct((_NPAD, 4 * NB), jnp.float32),
    )(node_base, d, m, dx, dy, dz, dst, gath, W_slice, b_slice, W2, b2)


def kernel(z, pos, emb, filter_W, filter_b, inter_W1, inter_b1, inter_W2,
           inter_b2, mix_W1, mix_b1, mix_W2, mix_b2, mu_mix_W):
    d2n = jnp.sum(pos * pos, axis=1)
    D2 = d2n[:, None] + d2n[None, :] - 2.0 * (pos @ pos.T)
    ar = jnp.arange(N)
    mask = (D2 < CUTOFF ** 2) & (ar[:, None] != ar[None, :])
    idx_i, idx_j = jnp.nonzero(mask, size=E_MAX, fill_value=0)
    edge_mask = jnp.arange(E_MAX) < jnp.sum(mask)
    m_f = edge_mask.astype(jnp.float32)

    r_ij = pos[idx_j] - pos[idx_i]
    r_ij = jnp.where(jnp.abs(r_ij) <= 1e-06, 1e-06, r_ij)
    d_ij = jnp.sqrt(jnp.sum(r_ij * r_ij, axis=1))
    dir_ij = r_ij / d_ij[:, None]
    dx, dy, dz = dir_ij[:, 0], dir_ij[:, 1], dir_ij[:, 2]

    dst = idx_i.astype(jnp.int32)
    src = idx_j.astype(jnp.int32)
    node_base = jnp.minimum((dst[::_T] // 8) * 8, _NPAD - _W).astype(jnp.int32)

    W_pad = jnp.zeros((_NRBF_PAD, NI * 3 * NB), jnp.float32)
    W_pad = W_pad.at[:NRBF].set(filter_W)

    q = emb[z]
    mu = jnp.zeros((N, 3 * NB), jnp.float32)
    for i in range(NI):
        s = jax.nn.silu(q @ inter_W1[i] + inter_b1[i])
        if i == 0:
            tab = lax.bitcast_convert_type(s, jnp.int32)
        else:
            tab = _pack_bf16(jnp.concatenate([s, mu], axis=1))
        gath = _sc_gather(tab, src)
        acc = _msg_pallas(i > 0, node_base, d_ij, m_f, dx, dy, dz, dst, gath,
                          W_pad[:, i * 3 * NB:(i + 1) * 3 * NB],
                          filter_b[i * 3 * NB:(i + 1) * 3 * NB],
                          inter_W2[i], inter_b2[i])
        q = q + acc[:N, :NB]
        mu = mu + acc[:N, NB:]
        mu3 = mu.reshape(N, 3, NB)
        mu_mix = mu3 @ mu_mix_W[i]
        mu_V, mu_Wc = jnp.split(mu_mix, 2, axis=-1)
        mu_Vn = jnp.sqrt(jnp.sum(mu_V ** 2, axis=-2) + EPS)
        ctx = jnp.concatenate([q, mu_Vn], axis=-1)
        xm = jax.nn.silu(ctx @ mix_W1[i] + mix_b1[i])
        xm = xm @ mix_W2[i] + mix_b2[i]
        dq_intra = xm[:, :NB]
        dmu_intra = xm[:, None, NB:2 * NB] * mu_Wc
        dqmu_intra = xm[:, 2 * NB:] * jnp.sum(mu_V * mu_Wc, axis=1)
        q = q + dq_intra + dqmu_intra
        mu = (mu3 + dmu_intra).reshape(N, 3 * NB)
    return q, mu.reshape(N, 3, NB)
